# baseline (speedup 1.0000x reference)
import jax
import jax.numpy as jnp
from jax.experimental import pallas as pl
from jax.experimental.pallas import tpu as pltpu

_NCHUNK = 256
_TBLK = 4


def _vq_kernel(z_ref, w_ref, out_ref, w2x_ref, w2c_ref):
    @pl.when(pl.program_id(0) == 0)
    def _():
        w = w_ref[...]
        w2x_ref[...] = -(w + w)
        w2c_ref[...] = jnp.sum(w * w, axis=1, keepdims=True)

    w2x = w2x_ref[...]
    w2c = w2c_ref[...]
    n = z_ref.shape[2]
    for tt in range(z_ref.shape[0]):
        zt = z_ref[tt]
        for c in range(0, n, _NCHUNK):
            nxw2 = jax.lax.dot_general(
                w2x, zt[:, c:c + _NCHUNK], (((1,), (0,)), ((), ())),
                preferred_element_type=jnp.float32,
                precision=jax.lax.Precision.DEFAULT)
            key = w2c + nxw2
            m = jnp.min(key, axis=0, keepdims=True)
            iota = jax.lax.broadcasted_iota(jnp.int32, key.shape, 0)
            cand = jnp.where(key == m, iota, key.shape[0])
            out_ref[tt, 0, c:c + _NCHUNK] = jnp.min(cand, axis=0).astype(jnp.int32)


def kernel(z, W):
    t, a, b, c = z.shape
    n = b * c
    k = W.shape[0]
    z3 = z.reshape(t, a, n)
    out = pl.pallas_call(
        _vq_kernel,
        grid=(t // _TBLK,),
        in_specs=[
            pl.BlockSpec((_TBLK, a, n), lambda i: (i, 0, 0)),
            pl.BlockSpec((k, a), lambda i: (0, 0)),
        ],
        out_specs=pl.BlockSpec((_TBLK, 1, n), lambda i: (i, 0, 0)),
        out_shape=jax.ShapeDtypeStruct((t, 1, n), jnp.int32),
        scratch_shapes=[
            pltpu.VMEM((k, a), jnp.float32),
            pltpu.VMEM((k, 1), jnp.float32),
        ],
    )(z3, W)
    return out.reshape(t, b, c)


# speedup vs baseline: 1.0182x; 1.0182x over previous
import jax
import jax.numpy as jnp
from jax.experimental import pallas as pl
from jax.experimental.pallas import tpu as pltpu

_NCHUNK = 256
_TBLK = 4
_APAD = 264


def _vq_kernel(z_ref, w_ref, out_ref, wa_ref, za_ref):
    @pl.when(pl.program_id(0) == 0)
    def _():
        w = w_ref[...]
        wa_ref[:, 0:256] = -(w + w)
        w2 = jnp.sum(w * w, axis=1, keepdims=True)
        h1 = w2.astype(jnp.bfloat16).astype(jnp.float32)
        r1 = w2 - h1
        h2 = r1.astype(jnp.bfloat16).astype(jnp.float32)
        h3 = r1 - h2
        wa_ref[:, 256:257] = h1
        wa_ref[:, 257:258] = h2
        wa_ref[:, 258:259] = h3
        wa_ref[:, 259:] = jnp.zeros_like(wa_ref[:, 259:])
        za_ref[256:259, :] = jnp.ones_like(za_ref[256:259, :])
        za_ref[259:, :] = jnp.zeros_like(za_ref[259:, :])

    wa = wa_ref[...]
    n = z_ref.shape[2]
    for tt in range(z_ref.shape[0]):
        za_ref[0:256, :] = z_ref[tt]
        za = za_ref[...]
        for c in range(0, n, _NCHUNK):
            key = jax.lax.dot_general(
                wa, za[:, c:c + _NCHUNK], (((1,), (0,)), ((), ())),
                preferred_element_type=jnp.float32,
                precision=jax.lax.Precision.DEFAULT)
            out_ref[tt, 0, c:c + _NCHUNK] = jnp.argmin(key, axis=0).astype(jnp.int32)


def kernel(z, W):
    t, a, b, c = z.shape
    n = b * c
    k = W.shape[0]
    z3 = z.reshape(t, a, n)
    out = pl.pallas_call(
        _vq_kernel,
        grid=(t // _TBLK,),
        in_specs=[
            pl.BlockSpec((_TBLK, a, n), lambda i: (i, 0, 0)),
            pl.BlockSpec((k, a), lambda i: (0, 0)),
        ],
        out_specs=pl.BlockSpec((_TBLK, 1, n), lambda i: (i, 0, 0)),
        out_shape=jax.ShapeDtypeStruct((t, 1, n), jnp.int32),
        scratch_shapes=[
            pltpu.VMEM((k, _APAD), jnp.float32),
            pltpu.VMEM((_APAD, n), jnp.float32),
        ],
    )(z3, W)
    return out.reshape(t, b, c)


# speedup vs baseline: 1.0641x; 1.0452x over previous
import jax
import jax.numpy as jnp
from jax.experimental import pallas as pl
from jax.experimental.pallas import tpu as pltpu

_NCHUNK = 128
_TBLK = 4


def _vq_kernel(z_ref, w_ref, out_ref, w2x_ref, w2c_ref):
    @pl.when(pl.program_id(0) == 0)
    def _():
        w = w_ref[...]
        w2x_ref[...] = -(w + w)
        w2c_ref[...] = jnp.sum(w * w, axis=1, keepdims=True)

    w2x = w2x_ref[...]
    w2c = w2c_ref[...]
    n = z_ref.shape[2]
    for tt in range(z_ref.shape[0]):
        zt = z_ref[tt]
        for c in range(0, n, _NCHUNK):
            nxw2 = jax.lax.dot_general(
                w2x, zt[:, c:c + _NCHUNK], (((1,), (0,)), ((), ())),
                preferred_element_type=jnp.float32,
                precision=jax.lax.Precision.DEFAULT)
            key = w2c + nxw2
            out_ref[tt, 0, c:c + _NCHUNK] = jnp.argmin(key, axis=0).astype(jnp.int32)


def kernel(z, W):
    t, a, b, c = z.shape
    n = b * c
    k = W.shape[0]
    z3 = z.reshape(t, a, n)
    out = pl.pallas_call(
        _vq_kernel,
        grid=(t // _TBLK,),
        in_specs=[
            pl.BlockSpec((_TBLK, a, n), lambda i: (i, 0, 0)),
            pl.BlockSpec((k, a), lambda i: (0, 0)),
        ],
        out_specs=pl.BlockSpec((_TBLK, 1, n), lambda i: (i, 0, 0)),
        out_shape=jax.ShapeDtypeStruct((t, 1, n), jnp.int32),
        scratch_shapes=[
            pltpu.VMEM((k, a), jnp.float32),
            pltpu.VMEM((k, 1), jnp.float32),
        ],
    )(z3, W)
    return out.reshape(t, b, c)


# speedup vs baseline: 1.3226x; 1.2428x over previous
import jax
import jax.numpy as jnp
from jax.experimental import pallas as pl
from jax.experimental.pallas import tpu as pltpu

_NCHUNK = 256
_TBLK = 4


def _vq_kernel(z_ref, w_ref, out_ref, w2x_ref, w2c_ref):
    @pl.when(pl.program_id(0) == 0)
    def _():
        w = w_ref[...]
        w2x_ref[...] = -(w + w)
        w2c_ref[...] = jnp.sum(w * w, axis=1, keepdims=True)

    w2x = w2x_ref[...]
    w2c = w2c_ref[...]
    n = z_ref.shape[2]
    for tt in range(z_ref.shape[0]):
        zt = z_ref[tt]
        for c in range(0, n, _NCHUNK):
            nxw2 = jax.lax.dot_general(
                w2x, zt[:, c:c + _NCHUNK], (((1,), (0,)), ((), ())),
                preferred_element_type=jnp.float32,
                precision=jax.lax.Precision.DEFAULT)
            key = w2c + nxw2
            out_ref[tt, 0, c:c + _NCHUNK] = jnp.argmin(key, axis=0).astype(jnp.int32)


def kernel(z, W):
    t, a, b, c = z.shape
    n = b * c
    k = W.shape[0]
    z3 = z.reshape(t, a, n)
    out = pl.pallas_call(
        _vq_kernel,
        grid=(t // _TBLK,),
        in_specs=[
            pl.BlockSpec((_TBLK, a, n), lambda i: (i, 0, 0)),
            pl.BlockSpec((k, a), lambda i: (0, 0)),
        ],
        out_specs=pl.BlockSpec((_TBLK, 1, n), lambda i: (i, 0, 0)),
        out_shape=jax.ShapeDtypeStruct((t, 1, n), jnp.int32),
        scratch_shapes=[
            pltpu.VMEM((k, a), jnp.float32),
            pltpu.VMEM((k, 1), jnp.float32),
        ],
    )(z3, W)
    return out.reshape(t, b, c)
